# Initial kernel scaffold; baseline (speedup 1.0000x reference)
import jax
import jax.numpy as jnp
from jax.experimental import pallas as pl


def kernel(x, edge_index, edge_weight, batch, W1, b1, W2, b2, W3, b3):
    raise NotImplementedError("write your pallas kernel here")


# speedup vs baseline: 27.7367x; 27.7367x over previous
import dataclasses
import functools

import jax
import jax.numpy as jnp
from jax import lax
from jax.experimental import pallas as pl
from jax.experimental.pallas import tpu as pltpu
from jax.experimental.pallas import tpu_sc as plsc

N = 10000
NPAD = 10240
E = 320000
EPAD = 327680
ER = EPAD // 128
F_IN = 128
H = 16
A = 8
G = 16

NCORE = 2
NSUB = 16
ROWS_PER_TILE = ER // (NCORE * NSUB)
DEG_ROWS_PER_TILE = ER // NSUB
NODE_SLICE = NPAD // NSUB
CHUNK_ROWS = 8
CHUNK_E = CHUNK_ROWS * 128

def _rsqrt16(x):
    i = plsc.bitcast(x, jnp.int32)
    i = jnp.int32(0x5F3759DF) - (i >> 1)
    y = plsc.bitcast(i, jnp.float32)
    for _ in range(3):
        y = y * (jnp.float32(1.5) - jnp.float32(0.5) * x * y * y)
    return y


def _splat(v):
    return jnp.full((16,), v, dtype=jnp.int32)


def _edge_pass(table2, acc, row_hbm, col_hbm, ew_hbm, rowb, colb, ewb, featb,
               ebase, iota):
    @pl.loop(0, ROWS_PER_TILE // CHUNK_ROWS)
    def _(ch):
        r0 = ebase + ch * CHUNK_ROWS
        pltpu.sync_copy(row_hbm.at[pl.ds(r0, CHUNK_ROWS)], rowb)
        pltpu.sync_copy(col_hbm.at[pl.ds(r0, CHUNK_ROWS)], colb)
        pltpu.sync_copy(ew_hbm.at[pl.ds(r0, CHUNK_ROWS)], ewb)
        for k in range(CHUNK_ROWS):
            pltpu.sync_copy(table2.at[rowb.at[k]],
                            featb.at[pl.ds(k * 128, 128)])

        @pl.loop(0, CHUNK_E, step=4)
        def _(j):
            for u in range(4):
                jj = j + u
                fj = _splat(jj)
                ew = plsc.load_gather(ewb, [_splat(jj >> 7), _splat(jj & 127)])
                r = plsc.load_gather(featb, [fj, iota])
                plsc.store_scatter(featb, [fj, iota], r * ew)

        for k in range(CHUNK_ROWS):
            pltpu.sync_copy(featb.at[pl.ds(k * 128, 128)],
                            acc.at[colb.at[k]], add=True)


def _out_pass(acc, part_out, stage, tbuf, dinvb, cid, nr0, iota):
    pltpu.sync_copy(acc.at[pl.ds(nr0, NODE_SLICE)], tbuf)
    fsel = jnp.float32(1.0) - cid.astype(jnp.float32)

    @pl.loop(0, NODE_SLICE, step=4)
    def _(j):
        for u in range(4):
            fj = _splat(j + u)
            r = plsc.load_gather(tbuf, [fj, iota])
            t = plsc.load_gather(stage, [fj, iota])
            d = plsc.load_gather(dinvb, [fj])
            plsc.store_scatter(tbuf, [fj, iota], (r + fsel * t) * d)

    pltpu.sync_copy(tbuf, part_out.at[cid, pl.ds(nr0, NODE_SLICE)])


def _sc_layer1_body(xw_hbm, row_hbm, col_hbm, ew_hbm, part_out, dinv_out,
                    table2, acc, deg, rowb, colb, ewb, featb, stage, tbuf,
                    degb, dinvb):
    cid = lax.axis_index("c")
    sid = lax.axis_index("s")
    iota = lax.iota(jnp.int32, 16)
    nr0 = sid * NODE_SLICE
    z16 = jnp.zeros((16,), jnp.float32)

    @pl.loop(0, NODE_SLICE, step=4)
    def _(j):
        for u in range(4):
            plsc.store_scatter(stage, [_splat(j + u), iota], z16)

    @pl.loop(0, NODE_SLICE // 16)
    def _(k):
        degb[pl.ds(k * 16, 16)] = z16

    pltpu.sync_copy(stage, acc.at[pl.ds(nr0, NODE_SLICE)])
    pltpu.sync_copy(degb, deg.at[pl.ds(nr0, NODE_SLICE)])
    plsc.subcore_barrier()

    @pl.loop(0, DEG_ROWS_PER_TILE // CHUNK_ROWS)
    def _(ch):
        r0 = sid * DEG_ROWS_PER_TILE + ch * CHUNK_ROWS
        pltpu.sync_copy(col_hbm.at[pl.ds(r0, CHUNK_ROWS)], colb)
        pltpu.sync_copy(ew_hbm.at[pl.ds(r0, CHUNK_ROWS)], ewb)
        for k in range(CHUNK_ROWS):
            pltpu.sync_copy(ewb.at[k], deg.at[colb.at[k]], add=True)

    plsc.subcore_barrier()

    pltpu.sync_copy(deg.at[pl.ds(nr0, NODE_SLICE)], degb)

    @pl.loop(0, NODE_SLICE // 16)
    def _(k):
        d = degb[pl.ds(k * 16, 16)] + jnp.float32(1.0)
        dinvb[pl.ds(k * 16, 16)] = _rsqrt16(d)

    @pl.when(cid == 0)
    def _():
        pltpu.sync_copy(dinvb, dinv_out.at[pl.ds(nr0, NODE_SLICE)])

    pltpu.sync_copy(xw_hbm.at[pl.ds(nr0, NODE_SLICE)], stage)

    @pl.loop(0, NODE_SLICE, step=4)
    def _(j):
        for u in range(4):
            fj = _splat(j + u)
            r = plsc.load_gather(stage, [fj, iota])
            d = plsc.load_gather(dinvb, [fj])
            plsc.store_scatter(stage, [fj, iota], r * d)

    pltpu.sync_copy(stage, table2.at[pl.ds(nr0, NODE_SLICE)])
    plsc.subcore_barrier()

    ebase = cid * (ER // NCORE) + sid * ROWS_PER_TILE
    _edge_pass(table2, acc, row_hbm, col_hbm, ew_hbm, rowb, colb, ewb,
               featb, ebase, iota)
    plsc.subcore_barrier()

    _out_pass(acc, part_out, stage, tbuf, dinvb, cid, nr0, iota)


def _sc_layer2_body(part_hbm, dinv_hbm, row_hbm, col_hbm, ew_hbm, b1_hbm,
                    part_out, table2, acc, rowb, colb, ewb, featb, stage,
                    tbuf, dinvb, biasb):
    cid = lax.axis_index("c")
    sid = lax.axis_index("s")
    iota = lax.iota(jnp.int32, 16)
    nr0 = sid * NODE_SLICE
    z16 = jnp.zeros((16,), jnp.float32)

    @pl.loop(0, NODE_SLICE, step=4)
    def _(j):
        for u in range(4):
            plsc.store_scatter(stage, [_splat(j + u), iota], z16)

    pltpu.sync_copy(stage, acc.at[pl.ds(nr0, NODE_SLICE)])
    pltpu.sync_copy(b1_hbm, biasb)
    pltpu.sync_copy(dinv_hbm.at[pl.ds(nr0, NODE_SLICE)], dinvb)
    pltpu.sync_copy(part_hbm.at[0, pl.ds(nr0, NODE_SLICE)], stage)
    pltpu.sync_copy(part_hbm.at[1, pl.ds(nr0, NODE_SLICE)], tbuf)
    bvec = biasb[...]

    @pl.loop(0, NODE_SLICE, step=4)
    def _(j):
        for u in range(4):
            fj = _splat(j + u)
            p0 = plsc.load_gather(stage, [fj, iota])
            p1 = plsc.load_gather(tbuf, [fj, iota])
            h = jnp.maximum(p0 + p1 + bvec, jnp.float32(0.0))
            d = plsc.load_gather(dinvb, [fj])
            plsc.store_scatter(stage, [fj, iota], h * d)

    pltpu.sync_copy(stage, table2.at[pl.ds(nr0, NODE_SLICE)])
    plsc.subcore_barrier()

    ebase = cid * (ER // NCORE) + sid * ROWS_PER_TILE
    _edge_pass(table2, acc, row_hbm, col_hbm, ew_hbm, rowb, colb, ewb,
               featb, ebase, iota)
    plsc.subcore_barrier()

    _out_pass(acc, part_out, stage, tbuf, dinvb, cid, nr0, iota)


@functools.cache
def _build_sc_kernels():
    mesh = plsc.VectorSubcoreMesh(core_axis_name="c", subcore_axis_name="s")
    cp = pltpu.CompilerParams()
    if "needs_layout_passes" in pltpu.CompilerParams.__dataclass_fields__:
        cp = dataclasses.replace(cp, needs_layout_passes=False,
                                 use_tc_tiling_on_sc=False)
    layer1 = pl.kernel(
        _sc_layer1_body,
        out_type=[
            jax.ShapeDtypeStruct((NCORE, NPAD, H), jnp.float32),
            jax.ShapeDtypeStruct((NPAD,), jnp.float32),
        ],
        mesh=mesh,
        scratch_types=[
            pltpu.VMEM_SHARED((NPAD, H), jnp.float32),
            pltpu.VMEM_SHARED((NPAD, H), jnp.float32),
            pltpu.VMEM_SHARED((NPAD,), jnp.float32),
            pltpu.VMEM((CHUNK_ROWS, 128), jnp.int32),
            pltpu.VMEM((CHUNK_ROWS, 128), jnp.int32),
            pltpu.VMEM((CHUNK_ROWS, 128), jnp.float32),
            pltpu.VMEM((CHUNK_E, H), jnp.float32),
            pltpu.VMEM((NODE_SLICE, H), jnp.float32),
            pltpu.VMEM((NODE_SLICE, H), jnp.float32),
            pltpu.VMEM((NODE_SLICE,), jnp.float32),
            pltpu.VMEM((NODE_SLICE,), jnp.float32),
        ],
        compiler_params=cp,
    )
    layer2 = pl.kernel(
        _sc_layer2_body,
        out_type=jax.ShapeDtypeStruct((NCORE, NPAD, H), jnp.float32),
        mesh=mesh,
        scratch_types=[
            pltpu.VMEM_SHARED((NPAD, H), jnp.float32),
            pltpu.VMEM_SHARED((NPAD, H), jnp.float32),
            pltpu.VMEM((CHUNK_ROWS, 128), jnp.int32),
            pltpu.VMEM((CHUNK_ROWS, 128), jnp.int32),
            pltpu.VMEM((CHUNK_ROWS, 128), jnp.float32),
            pltpu.VMEM((CHUNK_E, H), jnp.float32),
            pltpu.VMEM((NODE_SLICE, H), jnp.float32),
            pltpu.VMEM((NODE_SLICE, H), jnp.float32),
            pltpu.VMEM((NODE_SLICE,), jnp.float32),
            pltpu.VMEM((16,), jnp.float32),
        ],
        compiler_params=cp,
    )
    return layer1, layer2


def _mm_body(x_ref, w_ref, o_ref):
    o_ref[...] = jnp.dot(x_ref[...], w_ref[...],
                         preferred_element_type=jnp.float32)


_mm_call = pl.pallas_call(
    _mm_body,
    out_shape=jax.ShapeDtypeStruct((NPAD, H), jnp.float32),
)


def _epi_body(p_ref, b_ref, w2_ref, b2_ref, w3_ref, b3_ref, o_ref):
    s2 = p_ref[0] + p_ref[1]
    h2 = jnp.dot(s2, w2_ref[...], preferred_element_type=jnp.float32)
    h2 = jnp.maximum(h2 + b2_ref[...], 0.0)
    bt = b_ref[...]
    neg = jnp.float32(-jnp.inf)
    rows = []
    for g in range(G):
        m = jnp.where(bt == g, h2, neg)
        rows.append(jnp.max(m, axis=0, keepdims=True))
    pooled = jnp.concatenate(rows, axis=0)
    o_ref[...] = jnp.dot(pooled, w3_ref[...],
                         preferred_element_type=jnp.float32) + b3_ref[...]


_epi_call = pl.pallas_call(
    _epi_body,
    out_shape=jax.ShapeDtypeStruct((G, A), jnp.float32),
)


def kernel(x, edge_index, edge_weight, batch, W1, b1, W2, b2, W3, b3):
    row = edge_index[0]
    col = edge_index[1]
    padn = EPAD - E
    fill = (jnp.arange(padn, dtype=jnp.int32) % (NPAD - N)) + N
    rowp = jnp.concatenate([row, fill]).reshape(ER, 128)
    colp = jnp.concatenate([col, fill]).reshape(ER, 128)
    ewp = jnp.concatenate(
        [edge_weight, jnp.zeros((padn,), jnp.float32)]).reshape(ER, 128)
    xpad = jnp.pad(x, ((0, NPAD - N), (0, 0)))
    batchp = jnp.pad(batch, (0, NPAD - N), constant_values=G).reshape(NPAD, 1)

    sc_layer1, sc_layer2 = _build_sc_kernels()
    xw = _mm_call(xpad, W1)
    part, dinv = sc_layer1(xw, rowp, colp, ewp)
    part2 = sc_layer2(part, dinv, rowp, colp, ewp, b1)
    out = _epi_call(part2, batchp, W2, b2.reshape(1, H), W3, b3.reshape(1, A))
    return out


# speedup vs baseline: 34.6127x; 1.2479x over previous
import dataclasses
import functools

import jax
import jax.numpy as jnp
from jax import lax
from jax.experimental import pallas as pl
from jax.experimental.pallas import tpu as pltpu
from jax.experimental.pallas import tpu_sc as plsc

N = 10000
NPAD = 10240
E = 320000
EPAD = 327680
ER = EPAD // 128
F_IN = 128
H = 16
A = 8
G = 16

NCORE = 2
NSUB = 16
ROWS_PER_TILE = ER // (NCORE * NSUB)
DEG_ROWS_PER_TILE = ER // NSUB
NODE_SLICE = NPAD // NSUB
CHUNK_ROWS = 8
CHUNK_E = CHUNK_ROWS * 128
NCH = ROWS_PER_TILE // CHUNK_ROWS
NCH_DEG = DEG_ROWS_PER_TILE // CHUNK_ROWS


def _rsqrt16(x):
    i = plsc.bitcast(x, jnp.int32)
    i = jnp.int32(0x5F3759DF) - (i >> 1)
    y = plsc.bitcast(i, jnp.float32)
    for _ in range(3):
        y = y * (jnp.float32(1.5) - jnp.float32(0.5) * x * y * y)
    return y


def _splat(v):
    return jnp.full((16,), v, dtype=jnp.int32)


def _deg_pass(deg, col_hbm, ewf_hbm, colb, ewfb, sem_l, sem_s, sid):
    dbase = sid * DEG_ROWS_PER_TILE
    fbase = dbase * 128

    def loads(c, b):
        return [
            pltpu.async_copy(col_hbm.at[pl.ds(dbase + c * CHUNK_ROWS,
                                              CHUNK_ROWS)], colb[b], sem_l[b]),
            pltpu.async_copy(ewf_hbm.at[pl.ds(fbase + c * CHUNK_E, CHUNK_E)],
                             ewfb[b], sem_l[b]),
        ]

    lh = {0: loads(0, 0)}
    sh = {}
    for c in range(NCH_DEG):
        b = c % 2
        for x in lh.pop(c):
            x.wait()
        if c - 1 >= 0:
            for x in sh.pop(c - 1):
                x.wait()
        if c + 1 < NCH_DEG:
            lh[c + 1] = loads(c + 1, 1 - b)
        sh[c] = [
            pltpu.async_copy(ewfb[b].at[pl.ds(k * 128, 128)],
                             deg.at[colb[b].at[k]], sem_s[b], add=True)
            for k in range(CHUNK_ROWS)
        ]
    for x in sh[NCH_DEG - 1]:
        x.wait()


def _edge_pass(table2, acc, row_hbm, col_hbm, ewf_hbm, rowb, colb, ewfb,
               featb, sem_l, sem_g, sem_s, cid, sid, iota):
    ebase = cid * (ER // NCORE) + sid * ROWS_PER_TILE
    fbase = ebase * 128

    def loads(c, b):
        r0 = ebase + c * CHUNK_ROWS
        return [
            pltpu.async_copy(row_hbm.at[pl.ds(r0, CHUNK_ROWS)], rowb[b],
                             sem_l[b]),
            pltpu.async_copy(col_hbm.at[pl.ds(r0, CHUNK_ROWS)], colb[b],
                             sem_l[b]),
            pltpu.async_copy(ewf_hbm.at[pl.ds(fbase + c * CHUNK_E, CHUNK_E)],
                             ewfb[b], sem_l[b]),
        ]

    lh = {0: loads(0, 0)}
    gh = {}
    sh = {}
    for c in range(NCH):
        b = c % 2
        for x in lh.pop(c):
            x.wait()
        gh[c] = [
            pltpu.async_copy(table2.at[rowb[b].at[k]],
                             featb[b].at[pl.ds(k * 128, 128)], sem_g[b])
            for k in range(CHUNK_ROWS)
        ]
        if c - 1 >= 0:
            for x in sh.pop(c - 1):
                x.wait()
        if c + 1 < NCH:
            lh[c + 1] = loads(c + 1, 1 - b)
        for x in gh.pop(c):
            x.wait()

        fb = featb[b]
        eb = ewfb[b]

        @pl.loop(0, CHUNK_E, step=4)
        def _(j):
            for u in range(4):
                fj = _splat(j + u)
                ew = plsc.load_gather(eb, [fj])
                r = plsc.load_gather(fb, [fj, iota])
                plsc.store_scatter(fb, [fj, iota], r * ew)

        sh[c] = [
            pltpu.async_copy(featb[b].at[pl.ds(k * 128, 128)],
                             acc.at[colb[b].at[k]], sem_s[b], add=True)
            for k in range(CHUNK_ROWS)
        ]
    for x in sh[NCH - 1]:
        x.wait()


def _out_pass(acc, part_out, stage, tbuf, dinvb, cid, nr0, iota):
    pltpu.sync_copy(acc.at[pl.ds(nr0, NODE_SLICE)], tbuf)
    fsel = jnp.float32(1.0) - cid.astype(jnp.float32)

    @pl.loop(0, NODE_SLICE, step=4)
    def _(j):
        for u in range(4):
            fj = _splat(j + u)
            r = plsc.load_gather(tbuf, [fj, iota])
            t = plsc.load_gather(stage, [fj, iota])
            d = plsc.load_gather(dinvb, [fj])
            plsc.store_scatter(tbuf, [fj, iota], (r + fsel * t) * d)

    pltpu.sync_copy(tbuf, part_out.at[cid, pl.ds(nr0, NODE_SLICE)])


def _zero_stage(stage, iota):
    z16 = jnp.zeros((16,), jnp.float32)

    @pl.loop(0, NODE_SLICE, step=4)
    def _(j):
        for u in range(4):
            plsc.store_scatter(stage, [_splat(j + u), iota], z16)


def _sc_layer1_body(xw_hbm, row_hbm, col_hbm, ewf_hbm, part_out, dinv_out,
                    table2, acc, deg,
                    rowb0, rowb1, colb0, colb1, ewfb0, ewfb1, featb0, featb1,
                    stage, tbuf, degb, dinvb,
                    sem_l0, sem_l1, sem_g0, sem_g1, sem_s0, sem_s1):
    cid = lax.axis_index("c")
    sid = lax.axis_index("s")
    iota = lax.iota(jnp.int32, 16)
    nr0 = sid * NODE_SLICE
    rowb = (rowb0, rowb1)
    colb = (colb0, colb1)
    ewfb = (ewfb0, ewfb1)
    featb = (featb0, featb1)
    sem_l = (sem_l0, sem_l1)
    sem_g = (sem_g0, sem_g1)
    sem_s = (sem_s0, sem_s1)

    _zero_stage(stage, iota)

    @pl.loop(0, NODE_SLICE // 16)
    def _(k):
        degb[pl.ds(k * 16, 16)] = jnp.zeros((16,), jnp.float32)

    pltpu.sync_copy(stage, acc.at[pl.ds(nr0, NODE_SLICE)])
    pltpu.sync_copy(degb, deg.at[pl.ds(nr0, NODE_SLICE)])
    plsc.subcore_barrier()

    _deg_pass(deg, col_hbm, ewf_hbm, colb, ewfb, sem_l, sem_s, sid)
    plsc.subcore_barrier()

    pltpu.sync_copy(deg.at[pl.ds(nr0, NODE_SLICE)], degb)

    @pl.loop(0, NODE_SLICE // 16)
    def _(k):
        d = degb[pl.ds(k * 16, 16)] + jnp.float32(1.0)
        dinvb[pl.ds(k * 16, 16)] = _rsqrt16(d)

    @pl.when(cid == 0)
    def _():
        pltpu.sync_copy(dinvb, dinv_out.at[pl.ds(nr0, NODE_SLICE)])

    pltpu.sync_copy(xw_hbm.at[pl.ds(nr0, NODE_SLICE)], stage)

    @pl.loop(0, NODE_SLICE, step=4)
    def _(j):
        for u in range(4):
            fj = _splat(j + u)
            r = plsc.load_gather(stage, [fj, iota])
            d = plsc.load_gather(dinvb, [fj])
            plsc.store_scatter(stage, [fj, iota], r * d)

    pltpu.sync_copy(stage, table2.at[pl.ds(nr0, NODE_SLICE)])
    plsc.subcore_barrier()

    _edge_pass(table2, acc, row_hbm, col_hbm, ewf_hbm, rowb, colb, ewfb,
               featb, sem_l, sem_g, sem_s, cid, sid, iota)
    plsc.subcore_barrier()

    _out_pass(acc, part_out, stage, tbuf, dinvb, cid, nr0, iota)


def _sc_layer2_body(part_hbm, dinv_hbm, row_hbm, col_hbm, ewf_hbm, b1_hbm,
                    part_out, table2, acc,
                    rowb0, rowb1, colb0, colb1, ewfb0, ewfb1, featb0, featb1,
                    stage, tbuf, dinvb, biasb,
                    sem_l0, sem_l1, sem_g0, sem_g1, sem_s0, sem_s1):
    cid = lax.axis_index("c")
    sid = lax.axis_index("s")
    iota = lax.iota(jnp.int32, 16)
    nr0 = sid * NODE_SLICE
    rowb = (rowb0, rowb1)
    colb = (colb0, colb1)
    ewfb = (ewfb0, ewfb1)
    featb = (featb0, featb1)
    sem_l = (sem_l0, sem_l1)
    sem_g = (sem_g0, sem_g1)
    sem_s = (sem_s0, sem_s1)

    _zero_stage(stage, iota)
    pltpu.sync_copy(stage, acc.at[pl.ds(nr0, NODE_SLICE)])
    pltpu.sync_copy(b1_hbm, biasb)
    pltpu.sync_copy(dinv_hbm.at[pl.ds(nr0, NODE_SLICE)], dinvb)
    pltpu.sync_copy(part_hbm.at[0, pl.ds(nr0, NODE_SLICE)], stage)
    pltpu.sync_copy(part_hbm.at[1, pl.ds(nr0, NODE_SLICE)], tbuf)
    bvec = biasb[...]

    @pl.loop(0, NODE_SLICE, step=4)
    def _(j):
        for u in range(4):
            fj = _splat(j + u)
            p0 = plsc.load_gather(stage, [fj, iota])
            p1 = plsc.load_gather(tbuf, [fj, iota])
            h = jnp.maximum(p0 + p1 + bvec, jnp.float32(0.0))
            d = plsc.load_gather(dinvb, [fj])
            plsc.store_scatter(stage, [fj, iota], h * d)

    pltpu.sync_copy(stage, table2.at[pl.ds(nr0, NODE_SLICE)])
    plsc.subcore_barrier()

    _edge_pass(table2, acc, row_hbm, col_hbm, ewf_hbm, rowb, colb, ewfb,
               featb, sem_l, sem_g, sem_s, cid, sid, iota)
    plsc.subcore_barrier()

    _out_pass(acc, part_out, stage, tbuf, dinvb, cid, nr0, iota)


@functools.cache
def _build_sc_kernels():
    mesh = plsc.VectorSubcoreMesh(core_axis_name="c", subcore_axis_name="s")
    cp = pltpu.CompilerParams()
    if "needs_layout_passes" in pltpu.CompilerParams.__dataclass_fields__:
        cp = dataclasses.replace(cp, needs_layout_passes=False,
                                 use_tc_tiling_on_sc=False)
    common_scratch = [
        pltpu.VMEM((CHUNK_ROWS, 128), jnp.int32),
        pltpu.VMEM((CHUNK_ROWS, 128), jnp.int32),
        pltpu.VMEM((CHUNK_ROWS, 128), jnp.int32),
        pltpu.VMEM((CHUNK_ROWS, 128), jnp.int32),
        pltpu.VMEM((CHUNK_E,), jnp.float32),
        pltpu.VMEM((CHUNK_E,), jnp.float32),
        pltpu.VMEM((CHUNK_E, H), jnp.float32),
        pltpu.VMEM((CHUNK_E, H), jnp.float32),
        pltpu.VMEM((NODE_SLICE, H), jnp.float32),
        pltpu.VMEM((NODE_SLICE, H), jnp.float32),
    ]
    sems = [pltpu.SemaphoreType.DMA] * 6
    layer1 = pl.kernel(
        _sc_layer1_body,
        out_type=[
            jax.ShapeDtypeStruct((NCORE, NPAD, H), jnp.float32),
            jax.ShapeDtypeStruct((NPAD,), jnp.float32),
        ],
        mesh=mesh,
        scratch_types=[
            pltpu.VMEM_SHARED((NPAD, H), jnp.float32),
            pltpu.VMEM_SHARED((NPAD, H), jnp.float32),
            pltpu.VMEM_SHARED((NPAD,), jnp.float32),
        ] + common_scratch + [
            pltpu.VMEM((NODE_SLICE,), jnp.float32),
            pltpu.VMEM((NODE_SLICE,), jnp.float32),
        ] + sems,
        compiler_params=cp,
    )
    layer2 = pl.kernel(
        _sc_layer2_body,
        out_type=jax.ShapeDtypeStruct((NCORE, NPAD, H), jnp.float32),
        mesh=mesh,
        scratch_types=[
            pltpu.VMEM_SHARED((NPAD, H), jnp.float32),
            pltpu.VMEM_SHARED((NPAD, H), jnp.float32),
        ] + common_scratch + [
            pltpu.VMEM((NODE_SLICE,), jnp.float32),
            pltpu.VMEM((16,), jnp.float32),
        ] + sems,
        compiler_params=cp,
    )
    return layer1, layer2


def _mm_body(x_ref, w_ref, o_ref):
    o_ref[...] = jnp.dot(x_ref[...], w_ref[...],
                         preferred_element_type=jnp.float32)


_mm_call = pl.pallas_call(
    _mm_body,
    out_shape=jax.ShapeDtypeStruct((NPAD, H), jnp.float32),
)


def _epi_body(p_ref, b_ref, w2_ref, b2_ref, w3_ref, b3_ref, o_ref):
    s2 = p_ref[0] + p_ref[1]
    h2 = jnp.dot(s2, w2_ref[...], preferred_element_type=jnp.float32)
    h2 = jnp.maximum(h2 + b2_ref[...], 0.0)
    bt = b_ref[...]
    neg = jnp.float32(-jnp.inf)
    rows = []
    for g in range(G):
        m = jnp.where(bt == g, h2, neg)
        rows.append(jnp.max(m, axis=0, keepdims=True))
    pooled = jnp.concatenate(rows, axis=0)
    o_ref[...] = jnp.dot(pooled, w3_ref[...],
                         preferred_element_type=jnp.float32) + b3_ref[...]


_epi_call = pl.pallas_call(
    _epi_body,
    out_shape=jax.ShapeDtypeStruct((G, A), jnp.float32),
)


def kernel(x, edge_index, edge_weight, batch, W1, b1, W2, b2, W3, b3):
    row = edge_index[0]
    col = edge_index[1]
    padn = EPAD - E
    fill = (jnp.arange(padn, dtype=jnp.int32) % (NPAD - N)) + N
    rowp = jnp.concatenate([row, fill]).reshape(ER, 128)
    colp = jnp.concatenate([col, fill]).reshape(ER, 128)
    ewf = jnp.concatenate([edge_weight, jnp.zeros((padn,), jnp.float32)])
    xpad = jnp.pad(x, ((0, NPAD - N), (0, 0)))
    batchp = jnp.pad(batch, (0, NPAD - N), constant_values=G).reshape(NPAD, 1)

    sc_layer1, sc_layer2 = _build_sc_kernels()
    xw = _mm_call(xpad, W1)
    part, dinv = sc_layer1(xw, rowp, colp, ewf)
    part2 = sc_layer2(part, dinv, rowp, colp, ewf, b1)
    out = _epi_call(part2, batchp, W2, b2.reshape(1, H), W3, b3.reshape(1, A))
    return out
